# baseline (speedup 1.0000x reference)
import functools

import jax
import jax.numpy as jnp
from jax import lax
from jax.experimental import pallas as pl
from jax.experimental.pallas import tpu as pltpu
from jax.experimental.pallas import tpu_sc as plsc

_NC = 2
_NS = 16
_NW = _NC * _NS
_L = 16

_N = 8388608
_BLK = 128
_NB = _N // _BLK
_BLK_PER_W = _NB // _NW
_BCHUNK = 64
_NCHUNK = _BLK_PER_W // _BCHUNK


def _conf_body(lg_hbm, gt_hbm, out_hbm,
               lg_v0, lg_v1, gt_v0, gt_v1, res_v,
               sem_lg0, sem_lg1, sem_gt0, sem_gt1):
    cid = lax.axis_index("c")
    sid = lax.axis_index("s")
    wid = cid * _NS + sid
    base = wid * _BLK_PER_W

    zeros = jnp.zeros((_L,), jnp.int32)
    ones = jnp.ones((_L,), jnp.int32)

    lg_bufs = (lg_v0, lg_v1)
    gt_bufs = (gt_v0, gt_v1)
    sem_lg = (sem_lg0, sem_lg1)
    sem_gt = (sem_gt0, sem_gt1)

    def copies(c, b):
        boff = base + c * _BCHUNK
        h1 = pltpu.make_async_copy(
            lg_hbm.at[pl.ds(boff, _BCHUNK)], lg_bufs[b], sem_lg[b])
        h2 = pltpu.make_async_copy(
            gt_hbm.at[pl.ds(boff * _BLK, _BCHUNK * _BLK)], gt_bufs[b],
            sem_gt[b])
        return h1, h2

    def block_body(lg_b, gt_b, blk, accs2):
        a_p, a_g, a_pg = accs2
        for g in range(_BLK // _L):
            l0 = lg_b[blk, 0, pl.ds(g * _L, _L)]
            l1 = lg_b[blk, 1, pl.ds(g * _L, _L)]
            gt16 = gt_b[pl.ds(blk * _BLK + g * _L, _L)]
            pred = l1 > l0
            a_p = a_p + jnp.where(pred, ones, zeros)
            a_g = a_g + gt16
            a_pg = a_pg + jnp.where(pred, gt16, zeros)
        return (a_p, a_g, a_pg)

    for b in range(2):
        h1, h2 = copies(b, b)
        h1.start()
        h2.start()

    def pair_body(c2, accs):
        for b in range(2):
            c = c2 * 2 + b
            h1, h2 = copies(c, b)
            h1.wait()
            h2.wait()

            accs = lax.fori_loop(
                0, _BCHUNK,
                functools.partial(block_body, lg_bufs[b], gt_bufs[b]),
                accs, unroll=2)

            @pl.when(c + 2 < _NCHUNK)
            def _():
                n1, n2 = copies(c + 2, b)
                n1.start()
                n2.start()
        return accs

    z = jnp.zeros((_L,), jnp.int32)
    acc_p, acc_g, acc_pg = lax.fori_loop(
        0, _NCHUNK // 2, pair_body, (z, z, z))
    res_v[pl.ds(0, _L)] = acc_p
    res_v[pl.ds(_L, _L)] = acc_g
    res_v[pl.ds(2 * _L, _L)] = acc_pg
    pltpu.sync_copy(res_v, out_hbm.at[pl.ds(wid * 3 * _L, 3 * _L)])


_conf = functools.partial(
    pl.kernel,
    mesh=plsc.VectorSubcoreMesh(core_axis_name="c", subcore_axis_name="s"),
    out_type=jax.ShapeDtypeStruct((_NW * 3 * _L,), jnp.int32),
    compiler_params=pltpu.CompilerParams(needs_layout_passes=False),
    scratch_types=(
        [pltpu.VMEM((_BCHUNK, 2, _BLK), jnp.float32)] * 2
        + [pltpu.VMEM((_BCHUNK * _BLK,), jnp.int32)] * 2
        + [pltpu.VMEM((3 * _L,), jnp.int32)]
        + [pltpu.SemaphoreType.DMA] * 4
    ),
)(_conf_body)


def kernel(pred_logits, gt_labels):
    lg = pred_logits.reshape(_NB, _BLK, 2).transpose(0, 2, 1)
    parts = _conf(lg, gt_labels)
    s = parts.reshape(_NW, 3, _L).sum(axis=(0, 2))
    mix = jnp.array([[-1, -1, 1], [0, 1, -1], [1, 0, -1], [0, 0, 1]],
                    dtype=jnp.int32)
    off = jnp.array([pred_logits.shape[0], 0, 0, 0], dtype=jnp.int32)
    return (mix @ s + off).reshape(2, 2)


# speedup vs baseline: 1.1327x; 1.1327x over previous
import functools

import jax
import jax.numpy as jnp
from jax import lax
from jax.experimental import pallas as pl
from jax.experimental.pallas import tpu as pltpu
from jax.experimental.pallas import tpu_sc as plsc

_NC = 2
_NS = 16
_NW = _NC * _NS
_L = 16

_N = 8388608
_BLK = 128
_NB = _N // _BLK
_BLK_PER_W = _NB // _NW
_BCHUNK = 64
_NCHUNK = _BLK_PER_W // _BCHUNK


def _conf_body(lg_hbm, gt_hbm, out_hbm,
               lg_v0, lg_v1, lg_v2, lg_v3, gt_v0, gt_v1, gt_v2, gt_v3, res_v,
               sem_lg0, sem_lg1, sem_lg2, sem_lg3,
               sem_gt0, sem_gt1, sem_gt2, sem_gt3):
    cid = lax.axis_index("c")
    sid = lax.axis_index("s")
    wid = cid * _NS + sid
    base = wid * _BLK_PER_W

    zeros = jnp.zeros((_L,), jnp.int32)
    ones = jnp.ones((_L,), jnp.int32)

    lg_bufs = (lg_v0, lg_v1, lg_v2, lg_v3)
    gt_bufs = (gt_v0, gt_v1, gt_v2, gt_v3)
    sem_lg = (sem_lg0, sem_lg1, sem_lg2, sem_lg3)
    sem_gt = (sem_gt0, sem_gt1, sem_gt2, sem_gt3)

    def copies(c, b):
        boff = base + c * _BCHUNK
        h1 = pltpu.make_async_copy(
            lg_hbm.at[pl.ds(boff, _BCHUNK)], lg_bufs[b], sem_lg[b])
        h2 = pltpu.make_async_copy(
            gt_hbm.at[pl.ds(boff * _BLK, _BCHUNK * _BLK)], gt_bufs[b],
            sem_gt[b])
        return h1, h2

    def block_body(lg_b, gt_b, blk, accs2):
        a_p, a_g, a_pg = accs2
        for g in range(_BLK // _L):
            l0 = lg_b[blk, 0, pl.ds(g * _L, _L)]
            l1 = lg_b[blk, 1, pl.ds(g * _L, _L)]
            gt16 = gt_b[pl.ds(blk * _BLK + g * _L, _L)]
            pred = l1 > l0
            a_p = a_p + jnp.where(pred, ones, zeros)
            a_g = a_g + gt16
            a_pg = a_pg + jnp.where(pred, gt16, zeros)
        return (a_p, a_g, a_pg)

    for b in range(4):
        h1, h2 = copies(b, b)
        h1.start()
        h2.start()

    def quad_body(c4, accs):
        for b in range(4):
            c = c4 * 4 + b
            h1, h2 = copies(c, b)
            h1.wait()
            h2.wait()

            accs = lax.fori_loop(
                0, _BCHUNK,
                functools.partial(block_body, lg_bufs[b], gt_bufs[b]),
                accs, unroll=2)

            @pl.when(c + 4 < _NCHUNK)
            def _():
                n1, n2 = copies(c + 4, b)
                n1.start()
                n2.start()
        return accs

    z = jnp.zeros((_L,), jnp.int32)
    acc_p, acc_g, acc_pg = lax.fori_loop(
        0, _NCHUNK // 4, quad_body, (z, z, z))
    res_v[pl.ds(0, _L)] = acc_p
    res_v[pl.ds(_L, _L)] = acc_g
    res_v[pl.ds(2 * _L, _L)] = acc_pg
    pltpu.sync_copy(res_v, out_hbm.at[pl.ds(wid * 3 * _L, 3 * _L)])


_conf = functools.partial(
    pl.kernel,
    mesh=plsc.VectorSubcoreMesh(core_axis_name="c", subcore_axis_name="s"),
    out_type=jax.ShapeDtypeStruct((_NW * 3 * _L,), jnp.int32),
    compiler_params=pltpu.CompilerParams(needs_layout_passes=False),
    scratch_types=(
        [pltpu.VMEM((_BCHUNK, 2, _BLK), jnp.float32)] * 4
        + [pltpu.VMEM((_BCHUNK * _BLK,), jnp.int32)] * 4
        + [pltpu.VMEM((3 * _L,), jnp.int32)]
        + [pltpu.SemaphoreType.DMA] * 8
    ),
)(_conf_body)


def kernel(pred_logits, gt_labels):
    lg = pred_logits.reshape(_NB, _BLK, 2).transpose(0, 2, 1)
    parts = _conf(lg, gt_labels)
    s = parts.reshape(_NW, 3, _L).sum(axis=(0, 2))
    mix = jnp.array([[-1, -1, 1], [0, 1, -1], [1, 0, -1], [0, 0, 1]],
                    dtype=jnp.int32)
    off = jnp.array([pred_logits.shape[0], 0, 0, 0], dtype=jnp.int32)
    return (mix @ s + off).reshape(2, 2)


# speedup vs baseline: 1.1452x; 1.0110x over previous
import functools

import jax
import jax.numpy as jnp
from jax import lax
from jax.experimental import pallas as pl
from jax.experimental.pallas import tpu as pltpu
from jax.experimental.pallas import tpu_sc as plsc

_NC = 2
_NS = 16
_NW = _NC * _NS
_L = 16

_N = 8388608
_BLK = 128
_NB = _N // _BLK
_BLK_PER_W = _NB // _NW
_BCHUNK = 64
_NCHUNK = _BLK_PER_W // _BCHUNK


def _conf_body(lg_hbm, gt_hbm, out_hbm,
               lg_v0, lg_v1, lg_v2, lg_v3, gt_v0, gt_v1, gt_v2, gt_v3, res_v,
               sem_lg0, sem_lg1, sem_lg2, sem_lg3,
               sem_gt0, sem_gt1, sem_gt2, sem_gt3):
    cid = lax.axis_index("c")
    sid = lax.axis_index("s")
    wid = cid * _NS + sid
    base = wid * _BLK_PER_W

    zeros = jnp.zeros((_L,), jnp.int32)
    ones = jnp.ones((_L,), jnp.int32)

    lg_bufs = (lg_v0, lg_v1, lg_v2, lg_v3)
    gt_bufs = (gt_v0, gt_v1, gt_v2, gt_v3)
    sem_lg = (sem_lg0, sem_lg1, sem_lg2, sem_lg3)
    sem_gt = (sem_gt0, sem_gt1, sem_gt2, sem_gt3)

    def copies(c, b):
        boff = base + c * _BCHUNK
        h1 = pltpu.make_async_copy(
            lg_hbm.at[pl.ds(boff, _BCHUNK)], lg_bufs[b], sem_lg[b])
        h2 = pltpu.make_async_copy(
            gt_hbm.at[pl.ds(boff * _BLK, _BCHUNK * _BLK)], gt_bufs[b],
            sem_gt[b])
        return h1, h2

    def block_body(lg_b, gt_b, blk, accs2):
        a_p, a_g, a_pg = accs2
        for g in range(_BLK // _L):
            l0 = lg_b[blk, 0, pl.ds(g * _L, _L)]
            l1 = lg_b[blk, 1, pl.ds(g * _L, _L)]
            gt16 = gt_b[pl.ds(blk * _BLK + g * _L, _L)]
            pred = l1 > l0
            a_p = a_p + jnp.where(pred, ones, zeros)
            a_g = a_g + gt16
            a_pg = a_pg + jnp.where(pred, gt16, zeros)
        return (a_p, a_g, a_pg)

    for b in range(4):
        h1, h2 = copies(b, b)
        h1.start()
        h2.start()

    def quad_body(c4, accs):
        for b in range(4):
            c = c4 * 4 + b
            h1, h2 = copies(c, b)
            h1.wait()
            h2.wait()

            accs = lax.fori_loop(
                0, _BCHUNK,
                functools.partial(block_body, lg_bufs[b], gt_bufs[b]),
                accs, unroll=2)

            @pl.when(c + 4 < _NCHUNK)
            def _():
                n1, n2 = copies(c + 4, b)
                n1.start()
                n2.start()
        return accs

    z = jnp.zeros((_L,), jnp.int32)
    acc_p, acc_g, acc_pg = lax.fori_loop(
        0, _NCHUNK // 4, quad_body, (z, z, z))
    res_v[pl.ds(0, _L)] = acc_p
    res_v[pl.ds(_L, _L)] = acc_g
    res_v[pl.ds(2 * _L, _L)] = acc_pg
    pltpu.sync_copy(res_v, out_hbm.at[pl.ds(wid * 3 * _L, 3 * _L)])


_conf = functools.partial(
    pl.kernel,
    mesh=plsc.VectorSubcoreMesh(core_axis_name="c", subcore_axis_name="s"),
    out_type=jax.ShapeDtypeStruct((_NW * 3 * _L,), jnp.int32),
    compiler_params=pltpu.CompilerParams(needs_layout_passes=False),
    scratch_types=(
        [pltpu.VMEM((_BCHUNK, 2, _BLK), jnp.float32)] * 4
        + [pltpu.VMEM((_BCHUNK * _BLK,), jnp.int32)] * 4
        + [pltpu.VMEM((3 * _L,), jnp.int32)]
        + [pltpu.SemaphoreType.DMA] * 8
    ),
)(_conf_body)


def kernel(pred_logits, gt_labels):
    lg = pred_logits.reshape(_NB, _BLK, 2).transpose(0, 2, 1)
    parts = _conf(lg, gt_labels)
    mix = jnp.array([[-1, -1, 1], [0, 1, -1], [1, 0, -1], [0, 0, 1]],
                    dtype=jnp.float32)
    sel = (jnp.arange(_NW * 3 * _L) // _L) % 3
    amat = mix[:, sel]
    off = jnp.array([pred_logits.shape[0], 0, 0, 0], dtype=jnp.float32)
    conf = amat @ parts.astype(jnp.float32) + off
    return conf.astype(jnp.int32).reshape(2, 2)
